# baseline (device time: 1209639 ns/iter reference)
import jax
import jax.numpy as jnp
from jax import lax
from jax.experimental import pallas as pl
from jax.experimental.pallas import tpu as pltpu

NY = 4
CH = 1024


def kernel(partial, resid, gamma):
    _, M, D = partial.shape
    B = M // NY
    D2 = D // 2
    n_ch = B // CH
    gamma2d = gamma.reshape(1, D)

    def body(partial_ref, resid_ref, gamma_ref, out_ref,
             recv_r, recv_l, send_r, send_l,
             vw, vp, vr,
             copy_sems, rs_send, rs_recv, ag_send, ag_recv):
        my_x = lax.axis_index("x")
        my_y = lax.axis_index("y")
        my_z = lax.axis_index("z")
        right = (my_x, (my_y + 1) % NY, my_z)
        left = (my_x, (my_y + NY - 1) % NY, my_z)

        barrier = pltpu.get_barrier_semaphore()
        for nbr in (left, right):
            pl.semaphore_signal(barrier, inc=1, device_id=nbr,
                                device_id_type=pl.DeviceIdType.MESH)
        pl.semaphore_wait(barrier, 2)

        def rs_hop(s, src_half_r, src_half_l):
            rdma_r = pltpu.make_async_remote_copy(
                src_ref=src_half_r,
                dst_ref=recv_r.at[s],
                send_sem=rs_send.at[2 * s],
                recv_sem=rs_recv.at[2 * s],
                device_id=right,
                device_id_type=pl.DeviceIdType.MESH,
            )
            rdma_l = pltpu.make_async_remote_copy(
                src_ref=src_half_l,
                dst_ref=recv_l.at[s],
                send_sem=rs_send.at[2 * s + 1],
                recv_sem=rs_recv.at[2 * s + 1],
                device_id=left,
                device_id_type=pl.DeviceIdType.MESH,
            )
            rdma_r.start()
            rdma_l.start()
            rdma_r.wait()
            rdma_l.wait()

        b0r = (my_y + NY - 1) % NY
        b0l = (my_y + 1) % NY
        rs_hop(0,
               partial_ref.at[0, pl.ds(b0r * B, B), pl.ds(0, D2)],
               partial_ref.at[0, pl.ds(b0l * B, B), pl.ds(D2, D2)])

        for s in (1, 2):
            br = (my_y + NY - 1 - s) % NY
            bl = (my_y + 1 + s) % NY
            for c in range(n_ch):
                r0 = c * CH
                cps = [
                    pltpu.make_async_copy(
                        recv_r.at[s - 1, pl.ds(r0, CH), :],
                        vw.at[:, pl.ds(0, D2)], copy_sems.at[0]),
                    pltpu.make_async_copy(
                        recv_l.at[s - 1, pl.ds(r0, CH), :],
                        vw.at[:, pl.ds(D2, D2)], copy_sems.at[1]),
                    pltpu.make_async_copy(
                        partial_ref.at[0, pl.ds(br * B + r0, CH), pl.ds(0, D2)],
                        vp.at[:, pl.ds(0, D2)], copy_sems.at[2]),
                    pltpu.make_async_copy(
                        partial_ref.at[0, pl.ds(bl * B + r0, CH), pl.ds(D2, D2)],
                        vp.at[:, pl.ds(D2, D2)], copy_sems.at[3]),
                ]
                for cp in cps:
                    cp.start()
                for cp in cps:
                    cp.wait()
                vr[...] = vw[...] + vp[...]
                cpo_r = pltpu.make_async_copy(
                    vr.at[:, pl.ds(0, D2)],
                    send_r.at[s - 1, pl.ds(r0, CH), :], copy_sems.at[0])
                cpo_l = pltpu.make_async_copy(
                    vr.at[:, pl.ds(D2, D2)],
                    send_l.at[s - 1, pl.ds(r0, CH), :], copy_sems.at[1])
                cpo_r.start()
                cpo_l.start()
                cpo_r.wait()
                cpo_l.wait()
            rs_hop(s, send_r.at[s - 1], send_l.at[s - 1])

        for c in range(n_ch):
            r0 = my_y * B + c * CH
            cps = [
                pltpu.make_async_copy(
                    recv_r.at[2, pl.ds(c * CH, CH), :],
                    vw.at[:, pl.ds(0, D2)], copy_sems.at[0]),
                pltpu.make_async_copy(
                    recv_l.at[2, pl.ds(c * CH, CH), :],
                    vw.at[:, pl.ds(D2, D2)], copy_sems.at[1]),
                pltpu.make_async_copy(
                    partial_ref.at[0, pl.ds(r0, CH), :], vp, copy_sems.at[2]),
                pltpu.make_async_copy(
                    resid_ref.at[pl.ds(r0, CH), :], vr, copy_sems.at[3]),
            ]
            for cp in cps:
                cp.start()
            for cp in cps:
                cp.wait()
            y = vw[...] + vp[...] + vr[...]
            rms = jnp.sqrt(jnp.mean(y * y, axis=-1, keepdims=True) + 1e-6)
            vw[...] = y / rms * gamma_ref[...]
            cp_o = pltpu.make_async_copy(
                vw, out_ref.at[pl.ds(r0, CH), :], copy_sems.at[0])
            cp_o.start()
            cp_o.wait()

        for t in range(NY - 1):
            gr = (my_y + NY - t) % NY
            gl = (my_y + t) % NY
            rdma_r = pltpu.make_async_remote_copy(
                src_ref=out_ref.at[pl.ds(gr * B, B), pl.ds(0, D2)],
                dst_ref=out_ref.at[pl.ds(gr * B, B), pl.ds(0, D2)],
                send_sem=ag_send.at[2 * t],
                recv_sem=ag_recv.at[2 * t],
                device_id=right,
                device_id_type=pl.DeviceIdType.MESH,
            )
            rdma_l = pltpu.make_async_remote_copy(
                src_ref=out_ref.at[pl.ds(gl * B, B), pl.ds(D2, D2)],
                dst_ref=out_ref.at[pl.ds(gl * B, B), pl.ds(D2, D2)],
                send_sem=ag_send.at[2 * t + 1],
                recv_sem=ag_recv.at[2 * t + 1],
                device_id=left,
                device_id_type=pl.DeviceIdType.MESH,
            )
            rdma_r.start()
            rdma_l.start()
            rdma_r.wait()
            rdma_l.wait()

    out, _, _, _, _ = pl.pallas_call(
        body,
        out_shape=[
            jax.ShapeDtypeStruct((M, D), jnp.float32),
            jax.ShapeDtypeStruct((3, B, D2), jnp.float32),
            jax.ShapeDtypeStruct((3, B, D2), jnp.float32),
            jax.ShapeDtypeStruct((2, B, D2), jnp.float32),
            jax.ShapeDtypeStruct((2, B, D2), jnp.float32),
        ],
        in_specs=[
            pl.BlockSpec(memory_space=pl.ANY),
            pl.BlockSpec(memory_space=pl.ANY),
            pl.BlockSpec(memory_space=pltpu.MemorySpace.VMEM),
        ],
        out_specs=[pl.BlockSpec(memory_space=pl.ANY)] * 5,
        scratch_shapes=[
            pltpu.VMEM((CH, D), jnp.float32),
            pltpu.VMEM((CH, D), jnp.float32),
            pltpu.VMEM((CH, D), jnp.float32),
            pltpu.SemaphoreType.DMA((4,)),
            pltpu.SemaphoreType.DMA((6,)),
            pltpu.SemaphoreType.DMA((6,)),
            pltpu.SemaphoreType.DMA((6,)),
            pltpu.SemaphoreType.DMA((6,)),
        ],
        compiler_params=pltpu.CompilerParams(
            collective_id=0, vmem_limit_bytes=60 * 1024 * 1024),
    )(partial, resid, gamma2d)
    return out
